# baseline (speedup 1.0000x reference)
import jax
import jax.numpy as jnp
from jax.experimental import pallas as pl
from jax.experimental.pallas import tpu as pltpu

_TR = 512


def _gelu(v):
    return 0.5 * v * (1.0 + jax.lax.erf(v * (2.0 ** -0.5)))


def _stage_a(x_ref, pm_ref, lng_ref, lnb_ref, w1_ref, b1_ref, w2l_ref,
             l_ref, g_ref):
    t = pl.program_id(0)
    xv = x_ref[...]
    mu = jnp.mean(xv, axis=1, keepdims=True)
    var = jnp.mean((xv - mu) ** 2, axis=1, keepdims=True)
    vn = (xv - mu) / jnp.sqrt(var + 1e-5) * lng_ref[...] + lnb_ref[...]
    h1 = _gelu(jnp.dot(vn, w1_ref[...], preferred_element_type=jnp.float32)
               + b1_ref[...])
    c_half = h1.shape[1] // 2
    l_ref[...] = jnp.dot(h1[:, :c_half], w2l_ref[...],
                         preferred_element_type=jnp.float32)
    hg = h1[:, c_half:] * pm_ref[...]
    par = jax.lax.broadcasted_iota(jnp.int32, (xv.shape[0], 1), 0) % 2
    g0 = jnp.sum(jnp.where(par == 0, hg, 0.0), axis=0, keepdims=True)
    g1 = jnp.sum(jnp.where(par == 1, hg, 0.0), axis=0, keepdims=True)
    gm = jnp.concatenate([g0, g1], axis=0)

    @pl.when(t == 0)
    def _():
        g_ref[...] = gm

    @pl.when(t != 0)
    def _():
        g_ref[...] = g_ref[...] + gm


def _stage_b(l_ref, gm_ref, q_ref, w2g_ref, w2q_ref, b2_ref, w3_ref, b3_ref,
             w4p_ref, gd_ref, pm_ref, out_ref, qg_ref):
    t = pl.program_id(0)
    i = pl.program_id(1)
    nb = qg_ref.shape[0]
    B = gm_ref.shape[0]

    @pl.when(jnp.logical_and(t == 0, i == 0))
    def _():
        g_row = jnp.dot(gm_ref[...], w2g_ref[...],
                        preferred_element_type=jnp.float32)
        g_tiled = jnp.concatenate([g_row] * (nb // B), axis=0)
        q_row = jnp.dot(q_ref[...], w2q_ref[...],
                        preferred_element_type=jnp.float32)
        qg_ref[...] = g_tiled + q_row + b2_ref[...]

    rows = l_ref[...]
    qg0 = qg_ref[pl.ds(i * B, 1), :]
    qg1 = qg_ref[pl.ds(i * B + 1, 1), :]
    par = jax.lax.broadcasted_iota(jnp.int32, (rows.shape[0], 1), 0) % 2
    z2 = rows + jnp.where(par == 0, qg0, qg1)
    h2 = _gelu(z2)
    h3 = _gelu(jnp.dot(h2, w3_ref[...], preferred_element_type=jnp.float32)
               + b3_ref[...])
    logits = jnp.dot(h3, w4p_ref[...],
                     preferred_element_type=jnp.float32)
    delta = logits[:, 0:1] - logits[:, 1:2]
    post = jnp.where(delta + gd_ref[0] >= 0.0, 1.0, 0.0) * pm_ref[...]
    out_ref[0] = post


def kernel(x, query, pre_mask, pruning_index, ln_g, ln_b,
           W1, b1, W2, b2, W3, b3, W4, b4, gumbel):
    N, B, C = x.shape
    n = query.shape[1]
    ch = C // 2
    NR = N * B

    x2d = x.reshape(NR, C)
    q2d = query[-1].reshape(n * B, C)
    pm2d = jnp.transpose(pre_mask, (1, 0, 2)).reshape(NR, 1)
    w2l, w2g, w2q = W2[:ch], W2[ch:C], W2[C:]
    w4p = jnp.zeros((ch, 128), jnp.float32).at[:, :2].set(W4)
    gd = jnp.transpose(gumbel[..., 0] - gumbel[..., 1]
                       + (b4[0] - b4[1]), (1, 2, 0)).reshape(n, NR, 1)

    const2 = lambda shape: pl.BlockSpec(shape, lambda t: (0, 0))
    L, gsum = pl.pallas_call(
        _stage_a,
        grid=(NR // _TR,),
        in_specs=[
            pl.BlockSpec((_TR, C), lambda t: (t, 0)),
            pl.BlockSpec((_TR, 1), lambda t: (t, 0)),
            const2((1, C)), const2((1, C)),
            const2((C, C)), const2((1, C)),
            const2((ch, C)),
        ],
        out_specs=[
            pl.BlockSpec((_TR, C), lambda t: (t, 0)),
            pl.BlockSpec((B, ch), lambda t: (0, 0)),
        ],
        out_shape=[
            jax.ShapeDtypeStruct((NR, C), jnp.float32),
            jax.ShapeDtypeStruct((B, ch), jnp.float32),
        ],
    )(x2d, pm2d, ln_g.reshape(1, C), ln_b.reshape(1, C),
      W1, b1.reshape(1, C), w2l)

    psum = jnp.sum(pre_mask, axis=1)
    gmean = gsum / psum

    const3 = lambda shape: pl.BlockSpec(shape, lambda t, i: (0, 0))
    post = pl.pallas_call(
        _stage_b,
        grid=(NR // _TR, n),
        in_specs=[
            pl.BlockSpec((_TR, C), lambda t, i: (t, 0)),
            const3((B, ch)),
            const3((n * B, C)),
            const3((ch, C)), const3((C, C)), const3((1, C)),
            const3((C, ch)), const3((1, ch)),
            const3((ch, 128)),
            pl.BlockSpec((1, _TR, 1), lambda t, i: (i, t, 0)),
            pl.BlockSpec((_TR, 1), lambda t, i: (t, 0)),
        ],
        out_specs=pl.BlockSpec((1, _TR, 1), lambda t, i: (i, t, 0)),
        out_shape=jax.ShapeDtypeStruct((n, NR, 1), jnp.float32),
        scratch_shapes=[pltpu.VMEM((n * B, C), jnp.float32)],
    )(L, gmean, q2d, w2g, w2q, b2.reshape(1, C), W3, b3.reshape(1, ch),
      w4p, gd, pm2d)

    post_mask = jnp.transpose(post.reshape(n, N, B), (2, 0, 1))[..., None]
    loc = jnp.array([2, 3, 4, 5])
    ratio_train = jnp.array([0.6, 0.6, 0.3, 0.3], dtype=jnp.float32)
    gt = ratio_train[jnp.argmax(loc == pruning_index)]
    pred_ratio = jnp.mean(post_mask, axis=2)
    mask_loss = jnp.mean((pred_ratio - gt) ** 2, axis=1)
    return post_mask, mask_loss


# speedup vs baseline: 1.5067x; 1.5067x over previous
import jax
import jax.numpy as jnp
from jax.experimental import pallas as pl
from jax.experimental.pallas import tpu as pltpu

_TR = 1024
_LN = 128


def _gelu(v):
    return 0.5 * v * (1.0 + jax.lax.erf(v * (2.0 ** -0.5)))


def _stage_a(x_ref, pm_ref, lng_ref, lnb_ref, w1_ref, b1_ref, w2l_ref,
             l_ref, g_ref):
    t = pl.program_id(0)
    xv = x_ref[...]
    mu = jnp.mean(xv, axis=1, keepdims=True)
    var = jnp.mean((xv - mu) ** 2, axis=1, keepdims=True)
    vn = (xv - mu) / jnp.sqrt(var + 1e-5) * lng_ref[...] + lnb_ref[...]
    h1 = _gelu(jnp.dot(vn, w1_ref[...], preferred_element_type=jnp.float32)
               + b1_ref[...])
    c_half = h1.shape[1] // 2
    l_ref[...] = jnp.dot(h1[:, :c_half], w2l_ref[...],
                         preferred_element_type=jnp.float32)
    hg = h1[:, c_half:] * pm_ref[...]
    par = jax.lax.broadcasted_iota(jnp.int32, (xv.shape[0], 1), 0) % 2
    g0 = jnp.sum(jnp.where(par == 0, hg, 0.0), axis=0, keepdims=True)
    g1 = jnp.sum(jnp.where(par == 1, hg, 0.0), axis=0, keepdims=True)
    gm = jnp.concatenate([g0, g1], axis=0)

    @pl.when(t == 0)
    def _():
        g_ref[...] = gm

    @pl.when(t != 0)
    def _():
        g_ref[...] = g_ref[...] + gm


def _stage_b(l_ref, gm_ref, q_ref, w2g_ref, w2q_ref, b2_ref, w3_ref, b3_ref,
             w4p_ref, gd_ref, pm_ref, out_ref, qg_ref):
    t = pl.program_id(0)
    i = pl.program_id(1)
    nb = qg_ref.shape[0]
    B = gm_ref.shape[0]

    @pl.when(jnp.logical_and(t == 0, i == 0))
    def _():
        g_row = jnp.dot(gm_ref[...], w2g_ref[...],
                        preferred_element_type=jnp.float32)
        g_tiled = jnp.concatenate([g_row] * (nb // B), axis=0)
        q_row = jnp.dot(q_ref[...], w2q_ref[...],
                        preferred_element_type=jnp.float32)
        qg_ref[...] = g_tiled + q_row + b2_ref[...]

    rows = l_ref[...]
    qg0 = qg_ref[pl.ds(i * B, 1), :]
    qg1 = qg_ref[pl.ds(i * B + 1, 1), :]
    par = jax.lax.broadcasted_iota(jnp.int32, (rows.shape[0], 1), 0) % 2
    z2 = rows + jnp.where(par == 0, qg0, qg1)
    h2 = _gelu(z2)
    h3 = _gelu(jnp.dot(h2, w3_ref[...], preferred_element_type=jnp.float32)
               + b3_ref[...])
    logits = jnp.dot(h3, w4p_ref[...],
                     preferred_element_type=jnp.float32)
    delta = logits[:, 0:1] - logits[:, 1:2]
    dpk = jnp.reshape(delta, (delta.shape[0] // _LN, _LN))
    post = (jnp.where(dpk + gd_ref[0] >= 0.0, 1.0, 0.0) * pm_ref[...])
    out_ref[0] = post


def kernel(x, query, pre_mask, pruning_index, ln_g, ln_b,
           W1, b1, W2, b2, W3, b3, W4, b4, gumbel):
    N, B, C = x.shape
    n = query.shape[1]
    ch = C // 2
    NR = N * B
    npk = NR // _LN
    tpk = _TR // _LN

    x2d = x.reshape(NR, C)
    q2d = query[-1].reshape(n * B, C)
    pmr = jnp.transpose(pre_mask, (1, 0, 2)).reshape(NR, 1)
    pmpk = pmr.reshape(npk, _LN)
    w2l, w2g, w2q = W2[:ch], W2[ch:C], W2[C:]
    w4p = jnp.zeros((ch, 128), jnp.float32).at[:, :2].set(W4)
    gd = jnp.transpose(gumbel[..., 0] - gumbel[..., 1]
                       + (b4[0] - b4[1]), (1, 2, 0)).reshape(n, npk, _LN)

    const2 = lambda shape: pl.BlockSpec(shape, lambda t: (0, 0))
    L, gsum = pl.pallas_call(
        _stage_a,
        grid=(NR // _TR,),
        in_specs=[
            pl.BlockSpec((_TR, C), lambda t: (t, 0)),
            pl.BlockSpec((_TR, 1), lambda t: (t, 0)),
            const2((1, C)), const2((1, C)),
            const2((C, C)), const2((1, C)),
            const2((ch, C)),
        ],
        out_specs=[
            pl.BlockSpec((_TR, C), lambda t: (t, 0)),
            pl.BlockSpec((B, ch), lambda t: (0, 0)),
        ],
        out_shape=[
            jax.ShapeDtypeStruct((NR, C), jnp.float32),
            jax.ShapeDtypeStruct((B, ch), jnp.float32),
        ],
    )(x2d, pmr, ln_g.reshape(1, C), ln_b.reshape(1, C),
      W1, b1.reshape(1, C), w2l)

    psum = jnp.sum(pre_mask, axis=1)
    gmean = gsum / psum

    const3 = lambda shape: pl.BlockSpec(shape, lambda t, i: (0, 0))
    post = pl.pallas_call(
        _stage_b,
        grid=(NR // _TR, n),
        in_specs=[
            pl.BlockSpec((_TR, C), lambda t, i: (t, 0)),
            const3((B, ch)),
            const3((n * B, C)),
            const3((ch, C)), const3((C, C)), const3((1, C)),
            const3((C, ch)), const3((1, ch)),
            const3((ch, 128)),
            pl.BlockSpec((1, tpk, _LN), lambda t, i: (i, t, 0)),
            pl.BlockSpec((tpk, _LN), lambda t, i: (t, 0)),
        ],
        out_specs=pl.BlockSpec((1, tpk, _LN), lambda t, i: (i, t, 0)),
        out_shape=jax.ShapeDtypeStruct((n, npk, _LN), jnp.float32),
        scratch_shapes=[pltpu.VMEM((n * B, C), jnp.float32)],
    )(L, gmean, q2d, w2g, w2q, b2.reshape(1, C), W3, b3.reshape(1, ch),
      w4p, gd, pmpk)

    post_mask = jnp.transpose(post.reshape(n, N, B), (2, 0, 1))[..., None]
    loc = jnp.array([2, 3, 4, 5])
    ratio_train = jnp.array([0.6, 0.6, 0.3, 0.3], dtype=jnp.float32)
    gt = ratio_train[jnp.argmax(loc == pruning_index)]
    pr = jnp.sum(post.reshape(n, NR // 2, B), axis=1) / N
    pred_ratio = jnp.transpose(pr, (1, 0))[..., None]
    mask_loss = jnp.mean((pred_ratio - gt) ** 2, axis=1)
    return post_mask, mask_loss
